# baseline (speedup 1.0000x reference)
import functools

import jax
import jax.numpy as jnp
from jax import lax
from jax.experimental import pallas as pl
from jax.experimental.pallas import tpu as pltpu
from jax.experimental.pallas import tpu_sc as plsc

_HIDDEN = 64
_NW = 32
_CHUNK = 128
_NG = 6
_LOOKG = 3
_NS = 4


@functools.partial(jax.jit, static_argnames=("n_total",))
def _embed_gather(ids2d, table, n_total):
    n_per_w = n_total // _NW
    n_chunks = n_per_w // _CHUNK
    pairs_per_chunk = _CHUNK // 2
    mesh = plsc.VectorSubcoreMesh(core_axis_name="c", subcore_axis_name="s")

    @functools.partial(
        pl.kernel,
        out_type=jax.ShapeDtypeStruct((n_total // 2, 2 * _HIDDEN), jnp.float32),
        mesh=mesh,
        scratch_types=[
            pltpu.VMEM((n_chunks, _CHUNK), jnp.int32),
            pltpu.VMEM((_NG, _CHUNK, _HIDDEN), jnp.float32),
            pltpu.VMEM((_NS, pairs_per_chunk, 2 * _HIDDEN), jnp.float32),
            pltpu.SemaphoreType.DMA((_NG,)),
            pltpu.SemaphoreType.DMA((_NS,)),
        ],
        compiler_params=pltpu.CompilerParams(use_tc_tiling_on_sc=False),
    )
    def k(ids_hbm, table_hbm, out_hbm, idx_v, rows_v, pairs_v, gsem, ssem):
        wid = lax.axis_index("s") * 2 + lax.axis_index("c")
        row_base = wid * n_chunks
        pltpu.sync_copy(ids_hbm.at[pl.ds(row_base, n_chunks)], idx_v)

        def gather_desc(j, g):
            return pltpu.make_async_copy(
                table_hbm.at[idx_v.at[j]], rows_v.at[g], gsem.at[g])

        def store_desc(j, s):
            out_base = (row_base + j) * pairs_per_chunk
            return pltpu.make_async_copy(
                pairs_v.at[s],
                out_hbm.at[pl.ds(out_base, pairs_per_chunk)],
                ssem.at[s],
            )

        for j in range(_LOOKG):
            gather_desc(j, j % _NG).start()

        @pl.loop(0, n_chunks)
        def _(j):
            g = lax.rem(j, _NG)
            s = lax.rem(j, _NS)
            gather_desc(j, g).wait()

            @pl.when(j + _LOOKG < n_chunks)
            def _():
                gather_desc(j + _LOOKG, lax.rem(j + _LOOKG, _NG)).start()

            @pl.when(j >= _NS)
            def _():
                store_desc(j - _NS, s).wait()

            @pl.loop(0, pairs_per_chunk)
            def _(r2):
                for half in range(2):
                    for c in range(0, _HIDDEN, 16):
                        pairs_v[s, r2, pl.ds(half * _HIDDEN + c, 16)] = (
                            rows_v[g, 2 * r2 + half, pl.ds(c, 16)])

            store_desc(j, s).start()

        for j in range(n_chunks - _NS, n_chunks):
            store_desc(j, j % _NS).wait()

    return k(ids2d, table)


def kernel(input_ids, table):
    batch, seq = input_ids.shape
    n_total = batch * seq
    ids2d = input_ids.reshape(n_total // _CHUNK, _CHUNK)
    out = _embed_gather(ids2d, table, n_total)
    return out.reshape(batch, seq, _HIDDEN)


# speedup vs baseline: 1.2137x; 1.2137x over previous
import functools

import jax
import jax.numpy as jnp
from jax import lax
from jax.experimental import pallas as pl
from jax.experimental.pallas import tpu as pltpu
from jax.experimental.pallas import tpu_sc as plsc

_HIDDEN = 64
_NW = 32
_NBUF = 8
_LOOK = 4


@jax.jit
def _embed_gather(ids_in, table):
    batch, seq = ids_in.shape
    nb = batch // _NW
    h1 = 104
    h2 = seq - h1
    mesh = plsc.VectorSubcoreMesh(core_axis_name="c", subcore_axis_name="s")

    @functools.partial(
        pl.kernel,
        out_type=jax.ShapeDtypeStruct((batch, seq, _HIDDEN), jnp.float32),
        mesh=mesh,
        scratch_types=[
            pltpu.VMEM((nb, seq), jnp.int32),
            pltpu.VMEM((_NBUF, seq, _HIDDEN), jnp.float32),
            pltpu.SemaphoreType.DMA((_NBUF,)),
            pltpu.SemaphoreType.DMA((_NBUF,)),
        ],
        compiler_params=pltpu.CompilerParams(use_tc_tiling_on_sc=False),
    )
    def k(ids_hbm, table_hbm, out_hbm, idx_v, rows_v, gsem, ssem):
        wid = lax.axis_index("s") * 2 + lax.axis_index("c")
        b0 = wid * nb
        pltpu.sync_copy(ids_hbm.at[pl.ds(b0, nb), :], idx_v)

        def gather_descs(j, b):
            return (
                pltpu.make_async_copy(
                    table_hbm.at[idx_v.at[j, pl.ds(0, h1)]],
                    rows_v.at[b, pl.ds(0, h1)],
                    gsem.at[b],
                ),
                pltpu.make_async_copy(
                    table_hbm.at[idx_v.at[j, pl.ds(h1, h2)]],
                    rows_v.at[b, pl.ds(h1, h2)],
                    gsem.at[b],
                ),
            )

        def store_desc(j, b):
            return pltpu.make_async_copy(
                rows_v.at[b], out_hbm.at[b0 + j], ssem.at[b])

        for j in range(_LOOK):
            for d in gather_descs(j, j % _NBUF):
                d.start()

        def slot(j, b, first, last):
            for d in gather_descs(j, b):
                d.wait()
            store_desc(j, b).start()
            jn = j + _LOOK
            bn = (b + _LOOK) % _NBUF
            if not first:
                store_desc(jn - _NBUF, bn).wait()
            if not last:
                for d in gather_descs(jn, bn):
                    d.start()

        for b in range(_NBUF):
            slot(b, b, first=(b < _LOOK), last=False)

        @pl.loop(_NBUF, nb - _NBUF, step=_NBUF)
        def _(g):
            for b in range(_NBUF):
                slot(g + b, b, first=False, last=False)

        g_last = nb - _NBUF
        for b in range(_NBUF):
            slot(g_last + b, b, first=False, last=(b >= _NBUF - _LOOK))

        for b in range(_NBUF - _LOOK, _NBUF):
            store_desc(g_last + b, b).wait()

    return k(ids_in, table)


def kernel(input_ids, table):
    return _embed_gather(input_ids, table)


# speedup vs baseline: 1.2168x; 1.0025x over previous
import functools

import jax
import jax.numpy as jnp
from jax import lax
from jax.experimental import pallas as pl
from jax.experimental.pallas import tpu as pltpu
from jax.experimental.pallas import tpu_sc as plsc

_HIDDEN = 64
_NW = 32
_NBUF = 4
_LOOK = 2


@functools.partial(jax.jit, static_argnames=("seq",))
def _embed_gather(ids_in, table, seq):
    batch, seq_pad = ids_in.shape
    nb = batch // _NW
    h1 = 104
    h2 = seq - h1
    mesh = plsc.VectorSubcoreMesh(core_axis_name="c", subcore_axis_name="s")

    @functools.partial(
        pl.kernel,
        out_type=jax.ShapeDtypeStruct((batch, seq, _HIDDEN), jnp.float32),
        mesh=mesh,
        scratch_types=[
            pltpu.VMEM((nb, seq_pad), jnp.int32),
            pltpu.VMEM((_NBUF, seq, _HIDDEN), jnp.float32),
            pltpu.SemaphoreType.DMA((_NBUF,)),
            pltpu.SemaphoreType.DMA((_NBUF,)),
        ],
        compiler_params=pltpu.CompilerParams(use_tc_tiling_on_sc=False),
    )
    def k(ids_hbm, table_hbm, out_hbm, idx_v, rows_v, gsem, ssem):
        wid = lax.axis_index("s") * 2 + lax.axis_index("c")
        b0 = wid * nb
        pltpu.sync_copy(ids_hbm.at[pl.ds(b0, nb), :], idx_v)

        def gather_descs(j, b):
            return (
                pltpu.make_async_copy(
                    table_hbm.at[idx_v.at[j, pl.ds(0, h1)]],
                    rows_v.at[b, pl.ds(0, h1)],
                    gsem.at[b],
                ),
                pltpu.make_async_copy(
                    table_hbm.at[idx_v.at[j, pl.ds(h1, h2)]],
                    rows_v.at[b, pl.ds(h1, h2)],
                    gsem.at[b],
                ),
            )

        def store_desc(j, b):
            return pltpu.make_async_copy(
                rows_v.at[b], out_hbm.at[b0 + j], ssem.at[b])

        for j in range(_LOOK):
            for d in gather_descs(j, j % _NBUF):
                d.start()

        def slot(j, b, first, last):
            for d in gather_descs(j, b):
                d.wait()
            store_desc(j, b).start()
            jn = j + _LOOK
            bn = (b + _LOOK) % _NBUF
            if not first:
                store_desc(jn - _NBUF, bn).wait()
            if not last:
                for d in gather_descs(jn, bn):
                    d.start()

        for b in range(_NBUF):
            slot(b, b, first=(b < _LOOK), last=False)

        @pl.loop(_NBUF, nb - _NBUF, step=_NBUF)
        def _(g):
            for b in range(_NBUF):
                slot(g + b, b, first=False, last=False)

        g_last = nb - _NBUF
        for b in range(_NBUF):
            slot(g_last + b, b, first=False, last=(b >= _NBUF - _LOOK))

        for b in range(_NBUF - _LOOK, _NBUF):
            store_desc(g_last + b, b).wait()

    return k(ids_in, table)


def kernel(input_ids, table):
    batch, seq = input_ids.shape
    ids_pad = jnp.pad(input_ids, ((0, 0), (0, 256 - seq)))
    return _embed_gather(ids_pad, table, seq)
